# Initial kernel scaffold; baseline (speedup 1.0000x reference)
import functools

import jax
import jax.numpy as jnp
from jax import lax
from jax.experimental import pallas as pl
from jax.experimental.pallas import tpu as pltpu
from jax.experimental.pallas import tpu_sc as plsc

_B, _N, _C, _K = 16, 8192, 3, 512


def _fps_body(xt_ref, idx_ref, dist_ref):
    idx_ref[0:1, :] = jnp.zeros((1, _B), jnp.int32)
    dist_ref[...] = jnp.full((_B, _N), jnp.inf, jnp.float32)
    p0 = xt_ref[0, :, 0:1]
    p1 = xt_ref[1, :, 0:1]
    p2 = xt_ref[2, :, 0:1]

    def body(j, carry):
        p0, p1, p2 = carry
        x0 = xt_ref[0]
        x1 = xt_ref[1]
        x2 = xt_ref[2]
        t0 = x0 - p0
        t1 = x1 - p1
        t2 = x2 - p2
        d = t0 * t0 + t1 * t1 + t2 * t2
        dist = jnp.minimum(dist_ref[...], d)
        dist_ref[...] = dist
        m = jnp.max(dist, axis=1, keepdims=True)
        iota = lax.broadcasted_iota(jnp.int32, (_B, _N), 1)
        nxt = jnp.min(jnp.where(dist == m, iota, _N), axis=1)
        idx_ref[pl.ds(j, 1), :] = nxt[None, :]
        sel = iota == nxt[:, None]
        p0 = jnp.sum(jnp.where(sel, x0, 0.0), axis=1, keepdims=True)
        p1 = jnp.sum(jnp.where(sel, x1, 0.0), axis=1, keepdims=True)
        p2 = jnp.sum(jnp.where(sel, x2, 0.0), axis=1, keepdims=True)
        return (p0, p1, p2)

    lax.fori_loop(1, _K, body, (p0, p1, p2))


def _fps_indices(xt, interpret=False):
    return pl.pallas_call(
        _fps_body,
        out_shape=jax.ShapeDtypeStruct((_K, _B), jnp.int32),
        scratch_shapes=[pltpu.VMEM((_B, _N), jnp.float32)],
        interpret=interpret,
    )(xt)


_HALF = _K // 2


def _sc_gather_body(x_hbm, idx_hbm, out_hbm, xv, iv, ov):
    wid = lax.axis_index("s") * 2 + lax.axis_index("c")
    b = wid // 2
    h = wid % 2
    pltpu.sync_copy(x_hbm.at[b], xv)
    pltpu.sync_copy(idx_hbm.at[b, pl.ds(h * _HALF, _HALF)], iv)
    for t in range(_HALF // 16):
        ii = iv[pl.ds(t * 16, 16)]
        for c in range(_C):
            vals = plsc.load_gather(xv, [ii, jnp.full((16,), c, jnp.int32)])
            ov[c, pl.ds(t * 16, 16)] = vals
    pltpu.sync_copy(ov, out_hbm.at[b, h])


@jax.jit
def _sc_gather(x, idx):
    mesh = plsc.VectorSubcoreMesh(core_axis_name="c", subcore_axis_name="s")
    run = functools.partial(
        pl.kernel,
        mesh=mesh,
        out_type=jax.ShapeDtypeStruct((_B, 2, _C, _HALF), jnp.float32),
        scratch_types=[
            pltpu.VMEM((_N, _C), jnp.float32),
            pltpu.VMEM((_HALF,), jnp.int32),
            pltpu.VMEM((_C, _HALF), jnp.float32),
        ],
    )(_sc_gather_body)
    out4 = run(x, idx)
    return out4.transpose(0, 1, 3, 2).reshape(_B, _K, _C)


@jax.jit
def kernel(x):
    xt = jnp.transpose(x, (2, 0, 1))
    idx_kb = _fps_indices(xt)
    idx = idx_kb.T
    return _sc_gather(x, idx)


# speedup vs baseline: 24.6635x; 24.6635x over previous
import functools

import jax
import jax.numpy as jnp
from jax import lax
from jax.experimental import pallas as pl
from jax.experimental.pallas import tpu as pltpu
from jax.experimental.pallas import tpu_sc as plsc

_B, _N, _C, _K = 16, 8192, 3, 512


def _fps_body(xt_ref, idx_ref, dist_ref):
    idx_ref[0:1, :] = jnp.zeros((1, _B), jnp.int32)
    dist_ref[...] = jnp.full((_B, _N), jnp.inf, jnp.float32)
    p0 = xt_ref[0, :, 0:1]
    p1 = xt_ref[1, :, 0:1]
    p2 = xt_ref[2, :, 0:1]

    def body(j, carry):
        p0, p1, p2 = carry
        x0 = xt_ref[0]
        x1 = xt_ref[1]
        x2 = xt_ref[2]
        t0 = x0 - p0
        t1 = x1 - p1
        t2 = x2 - p2
        d = t0 * t0 + t1 * t1 + t2 * t2
        dist = jnp.minimum(dist_ref[...], d)
        dist_ref[...] = dist
        m = jnp.max(dist, axis=1, keepdims=True)
        iota = lax.broadcasted_iota(jnp.int32, (_B, _N), 1)
        nxt = jnp.min(jnp.where(dist == m, iota, _N), axis=1)
        idx_ref[pl.ds(j, 1), :] = nxt[None, :]
        sel = iota == nxt[:, None]
        p0 = jnp.sum(jnp.where(sel, x0, 0.0), axis=1, keepdims=True)
        p1 = jnp.sum(jnp.where(sel, x1, 0.0), axis=1, keepdims=True)
        p2 = jnp.sum(jnp.where(sel, x2, 0.0), axis=1, keepdims=True)
        return (p0, p1, p2)

    lax.fori_loop(1, _K, body, (p0, p1, p2))


def _fps_indices(xt, interpret=False):
    return pl.pallas_call(
        _fps_body,
        out_shape=jax.ShapeDtypeStruct((_K, _B), jnp.int32),
        scratch_shapes=[pltpu.VMEM((_B, _N), jnp.float32)],
        interpret=interpret,
    )(xt)


_HALF = _K // 2


def _sc_gather_body(x_hbm, idx_hbm, out_hbm, xv, iv, ov):
    wid = lax.axis_index("s") * 2 + lax.axis_index("c")
    b = wid // 2
    h = wid % 2
    pltpu.sync_copy(x_hbm.at[b], xv)
    pltpu.sync_copy(idx_hbm.at[b, pl.ds(h * _HALF, _HALF)], iv)
    for t in range(_HALF // 16):
        base = iv[pl.ds(t * 16, 16)] * 3
        for c in range(_C):
            vals = plsc.load_gather(xv, [base + c])
            ov[c, pl.ds(t * 16, 16)] = vals
    pltpu.sync_copy(ov, out_hbm.at[b, h])


@jax.jit
def _sc_gather(xflat, idx):
    mesh = plsc.VectorSubcoreMesh(core_axis_name="c", subcore_axis_name="s")
    run = functools.partial(
        pl.kernel,
        mesh=mesh,
        out_type=jax.ShapeDtypeStruct((_B, 2, _C, _HALF), jnp.float32),
        scratch_types=[
            pltpu.VMEM((_N * _C,), jnp.float32),
            pltpu.VMEM((_HALF,), jnp.int32),
            pltpu.VMEM((_C, _HALF), jnp.float32),
        ],
        compiler_params=pltpu.CompilerParams(needs_layout_passes=False),
    )(_sc_gather_body)
    out4 = run(xflat, idx)
    return out4.transpose(0, 1, 3, 2).reshape(_B, _K, _C)


@jax.jit
def kernel(x):
    xt = jnp.transpose(x, (2, 0, 1))
    idx_kb = _fps_indices(xt)
    idx = idx_kb.T
    return _sc_gather(x.reshape(_B, _N * _C), idx)


# speedup vs baseline: 32.5751x; 1.3208x over previous
import functools

import jax
import jax.numpy as jnp
from jax import lax
from jax.experimental import pallas as pl
from jax.experimental.pallas import tpu as pltpu
from jax.experimental.pallas import tpu_sc as plsc

_B, _N, _C, _K = 16, 8192, 3, 512


_LN = 128
_CH = _N // _LN
_NS = 2
_CPS = _CH // _NS


def _fps_body(xt_ref, idx_ref, dist_ref):
    idx_ref[0:1, :] = jnp.zeros((1, _B), jnp.int32)
    dist_ref[...] = jnp.full((_B, _N), jnp.inf, jnp.float32)
    p0 = xt_ref[0, :, 0:1]
    p1 = xt_ref[1, :, 0:1]
    p2 = xt_ref[2, :, 0:1]
    lane = lax.broadcasted_iota(jnp.int32, (_B, _LN), 1)

    def body(j, carry):
        p0, p1, p2 = carry

        def chunk_vals(c):
            sl = pl.ds(c * _LN, _LN)
            x0 = xt_ref[0, :, sl]
            x1 = xt_ref[1, :, sl]
            x2 = xt_ref[2, :, sl]
            t0 = x0 - p0
            t1 = x1 - p1
            t2 = x2 - p2
            d = t0 * t0 + t1 * t1 + t2 * t2
            nd = jnp.minimum(dist_ref[:, sl], d)
            dist_ref[:, sl] = nd
            return nd, x0, x1, x2

        accs = []
        for s in range(_NS):
            c0 = s * _CPS
            nd, x0, x1, x2 = chunk_vals(c0)
            bv, bc = nd, jnp.full((_B, _LN), c0, jnp.int32)
            b0, b1, b2 = x0, x1, x2
            for t in range(1, _CPS):
                c = c0 + t
                nd, x0, x1, x2 = chunk_vals(c)
                upd = nd > bv
                bv = jnp.where(upd, nd, bv)
                bc = jnp.where(upd, c, bc)
                b0 = jnp.where(upd, x0, b0)
                b1 = jnp.where(upd, x1, b1)
                b2 = jnp.where(upd, x2, b2)
            accs.append((bv, bc, b0, b1, b2))

        def merge(a, b):
            upd = b[0] > a[0]
            return tuple(jnp.where(upd, yb, ya) for ya, yb in zip(a, b))

        while len(accs) > 1:
            accs = [merge(accs[i], accs[i + 1]) if i + 1 < len(accs)
                    else accs[i] for i in range(0, len(accs), 2)]
        bv, bc, b0, b1, b2 = accs[0]
        n_cand = bc * _LN + lane
        m = jnp.max(bv, axis=1, keepdims=True)
        nxt = jnp.min(jnp.where(bv == m, n_cand, _N),
                      axis=1, keepdims=True)
        sel = n_cand == nxt
        p0 = jnp.sum(jnp.where(sel, b0, 0.0), axis=1, keepdims=True)
        p1 = jnp.sum(jnp.where(sel, b1, 0.0), axis=1, keepdims=True)
        p2 = jnp.sum(jnp.where(sel, b2, 0.0), axis=1, keepdims=True)
        idx_ref[pl.ds(j, 1), :] = nxt.reshape(1, _B)
        return (p0, p1, p2)

    lax.fori_loop(1, _K, body, (p0, p1, p2))


def _fps_indices(xt, interpret=False):
    return pl.pallas_call(
        _fps_body,
        out_shape=jax.ShapeDtypeStruct((_K, _B), jnp.int32),
        scratch_shapes=[pltpu.VMEM((_B, _N), jnp.float32)],
        interpret=interpret,
    )(xt)


_HALF = _K // 2


def _sc_gather_body(x_hbm, idx_hbm, out_hbm, xv, iv, ov):
    wid = lax.axis_index("s") * 2 + lax.axis_index("c")
    b = wid // 2
    h = wid % 2
    pltpu.sync_copy(x_hbm.at[b], xv)
    pltpu.sync_copy(idx_hbm.at[b, pl.ds(h * _HALF, _HALF)], iv)
    for t in range(_HALF // 16):
        base = iv[pl.ds(t * 16, 16)] * 3
        for c in range(_C):
            vals = plsc.load_gather(xv, [base + c])
            ov[c, pl.ds(t * 16, 16)] = vals
    pltpu.sync_copy(ov, out_hbm.at[b, h])


@jax.jit
def _sc_gather(xflat, idx):
    mesh = plsc.VectorSubcoreMesh(core_axis_name="c", subcore_axis_name="s")
    run = functools.partial(
        pl.kernel,
        mesh=mesh,
        out_type=jax.ShapeDtypeStruct((_B, 2, _C, _HALF), jnp.float32),
        scratch_types=[
            pltpu.VMEM((_N * _C,), jnp.float32),
            pltpu.VMEM((_HALF,), jnp.int32),
            pltpu.VMEM((_C, _HALF), jnp.float32),
        ],
        compiler_params=pltpu.CompilerParams(needs_layout_passes=False),
    )(_sc_gather_body)
    out4 = run(xflat, idx)
    return out4.transpose(0, 1, 3, 2).reshape(_B, _K, _C)


@jax.jit
def kernel(x):
    xt = jnp.transpose(x, (2, 0, 1))
    idx_kb = _fps_indices(xt)
    idx = idx_kb.T
    return _sc_gather(x.reshape(_B, _N * _C), idx)


# speedup vs baseline: 34.0520x; 1.0453x over previous
import functools

import jax
import jax.numpy as jnp
from jax import lax
from jax.experimental import pallas as pl
from jax.experimental.pallas import tpu as pltpu
from jax.experimental.pallas import tpu_sc as plsc

_B, _N, _C, _K = 16, 8192, 3, 512


_LN = 128
_CH = _N // _LN
_NS = 2
_CPS = _CH // _NS


def _fps_body(xt_ref, idx_ref, dist_ref):
    idx_ref[0:1, :] = jnp.zeros((1, _B), jnp.int32)
    dist_ref[...] = jnp.full((_B, _N), jnp.inf, jnp.float32)
    p0 = xt_ref[0, :, 0:1]
    p1 = xt_ref[1, :, 0:1]
    p2 = xt_ref[2, :, 0:1]
    lane = lax.broadcasted_iota(jnp.int32, (_B, _LN), 1)

    def body(j, carry):
        p0, p1, p2 = carry

        def chunk_vals(c):
            sl = pl.ds(c * _LN, _LN)
            x0 = xt_ref[0, :, sl]
            x1 = xt_ref[1, :, sl]
            x2 = xt_ref[2, :, sl]
            t0 = x0 - p0
            t1 = x1 - p1
            t2 = x2 - p2
            d = t0 * t0 + t1 * t1 + t2 * t2
            nd = jnp.minimum(dist_ref[:, sl], d)
            dist_ref[:, sl] = nd
            return nd, x0, x1, x2

        accs = []
        for s in range(_NS):
            c0 = s * _CPS
            nd, x0, x1, x2 = chunk_vals(c0)
            bv, bc = nd, jnp.full((_B, _LN), c0, jnp.int32)
            b0, b1, b2 = x0, x1, x2
            for t in range(1, _CPS):
                c = c0 + t
                nd, x0, x1, x2 = chunk_vals(c)
                upd = nd > bv
                bv = jnp.where(upd, nd, bv)
                bc = jnp.where(upd, c, bc)
                b0 = jnp.where(upd, x0, b0)
                b1 = jnp.where(upd, x1, b1)
                b2 = jnp.where(upd, x2, b2)
            accs.append((bv, bc, b0, b1, b2))

        def merge(a, b):
            upd = b[0] > a[0]
            return tuple(jnp.where(upd, yb, ya) for ya, yb in zip(a, b))

        while len(accs) > 1:
            accs = [merge(accs[i], accs[i + 1]) if i + 1 < len(accs)
                    else accs[i] for i in range(0, len(accs), 2)]
        bv, bc, b0, b1, b2 = accs[0]

        m = jnp.max(bv, axis=1, keepdims=True)
        eq = bv == m
        n_cand = bc * _LN + lane
        base = n_cand << 16

        def keymin(payload):
            k = jnp.where(eq, base | payload, jnp.int32(0x7FFFFFFF))
            return jnp.min(k, axis=1, keepdims=True)

        def halves(v):
            vb = lax.bitcast_convert_type(v, jnp.int32)
            return (vb >> 16) & jnp.int32(0xFFFF), vb & jnp.int32(0xFFFF)

        h0, l0 = halves(b0)
        h1, l1 = halves(b1)
        h2, l2 = halves(b2)
        k0h, k0l = keymin(h0), keymin(l0)
        k1h, k1l = keymin(h1), keymin(l1)
        k2h, k2l = keymin(h2), keymin(l2)

        def coord(kh, kl):
            vb = ((kh & jnp.int32(0xFFFF)) << 16) | (kl & jnp.int32(0xFFFF))
            return lax.bitcast_convert_type(vb, jnp.float32)

        p0, p1, p2 = coord(k0h, k0l), coord(k1h, k1l), coord(k2h, k2l)
        nxt = k0h >> 16
        idx_ref[pl.ds(j, 1), :] = nxt.reshape(1, _B)
        return (p0, p1, p2)

    lax.fori_loop(1, _K, body, (p0, p1, p2))


def _fps_indices(xt, interpret=False):
    return pl.pallas_call(
        _fps_body,
        out_shape=jax.ShapeDtypeStruct((_K, _B), jnp.int32),
        scratch_shapes=[pltpu.VMEM((_B, _N), jnp.float32)],
        interpret=interpret,
    )(xt)


_HALF = _K // 2


def _sc_gather_body(x_hbm, idx_hbm, out_hbm, xv, iv, ov):
    wid = lax.axis_index("s") * 2 + lax.axis_index("c")
    b = wid // 2
    h = wid % 2
    pltpu.sync_copy(x_hbm.at[b], xv)
    pltpu.sync_copy(idx_hbm.at[b, pl.ds(h * _HALF, _HALF)], iv)
    for t in range(_HALF // 16):
        base = iv[pl.ds(t * 16, 16)] * 3
        for c in range(_C):
            vals = plsc.load_gather(xv, [base + c])
            ov[c, pl.ds(t * 16, 16)] = vals
    pltpu.sync_copy(ov, out_hbm.at[b, h])


@jax.jit
def _sc_gather(xflat, idx):
    mesh = plsc.VectorSubcoreMesh(core_axis_name="c", subcore_axis_name="s")
    run = functools.partial(
        pl.kernel,
        mesh=mesh,
        out_type=jax.ShapeDtypeStruct((_B, 2, _C, _HALF), jnp.float32),
        scratch_types=[
            pltpu.VMEM((_N * _C,), jnp.float32),
            pltpu.VMEM((_HALF,), jnp.int32),
            pltpu.VMEM((_C, _HALF), jnp.float32),
        ],
        compiler_params=pltpu.CompilerParams(needs_layout_passes=False),
    )(_sc_gather_body)
    out4 = run(xflat, idx)
    return out4.transpose(0, 1, 3, 2).reshape(_B, _K, _C)


@jax.jit
def kernel(x):
    xt = jnp.transpose(x, (2, 0, 1))
    idx_kb = _fps_indices(xt)
    idx = idx_kb.T
    return _sc_gather(x.reshape(_B, _N * _C), idx)


# speedup vs baseline: 42.3310x; 1.2431x over previous
import functools

import jax
import jax.numpy as jnp
from jax import lax
from jax.experimental import pallas as pl
from jax.experimental.pallas import tpu as pltpu
from jax.experimental.pallas import tpu_sc as plsc

_B, _N, _C, _K = 16, 8192, 3, 512


_LN = 128
_CH = _N // _LN
_NS = 2
_CPS = _CH // _NS


def _fps_body(xt_ref, idx_ref, dist_ref):
    idx_ref[0:1, :] = jnp.zeros((1, _B), jnp.int32)
    dist_ref[...] = jnp.full((_B, _N), jnp.inf, jnp.float32)
    p0 = jnp.broadcast_to(xt_ref[0, :, 0:1], (_B, _LN))
    p1 = jnp.broadcast_to(xt_ref[1, :, 0:1], (_B, _LN))
    p2 = jnp.broadcast_to(xt_ref[2, :, 0:1], (_B, _LN))
    lane = lax.broadcasted_iota(jnp.int32, (_B, _LN), 1)

    def body(j, carry):
        p0, p1, p2 = carry

        def chunk_vals(c):
            sl = pl.ds(c * _LN, _LN)
            x0 = xt_ref[0, :, sl]
            x1 = xt_ref[1, :, sl]
            x2 = xt_ref[2, :, sl]
            t0 = x0 - p0
            t1 = x1 - p1
            t2 = x2 - p2
            d = t0 * t0 + t1 * t1 + t2 * t2
            nd = jnp.minimum(dist_ref[:, sl], d)
            dist_ref[:, sl] = nd
            return nd, x0, x1, x2

        accs = []
        for s in range(_NS):
            c0 = s * _CPS
            nd, x0, x1, x2 = chunk_vals(c0)
            bv, bc = nd, jnp.full((_B, _LN), c0, jnp.int32)
            b0, b1, b2 = x0, x1, x2
            for t in range(1, _CPS):
                c = c0 + t
                nd, x0, x1, x2 = chunk_vals(c)
                upd = nd > bv
                bv = jnp.where(upd, nd, bv)
                bc = jnp.where(upd, c, bc)
                b0 = jnp.where(upd, x0, b0)
                b1 = jnp.where(upd, x1, b1)
                b2 = jnp.where(upd, x2, b2)
            accs.append((bv, bc, b0, b1, b2))

        def merge(a, b):
            upd = b[0] > a[0]
            return tuple(jnp.where(upd, yb, ya) for ya, yb in zip(a, b))

        while len(accs) > 1:
            accs = [merge(accs[i], accs[i + 1]) if i + 1 < len(accs)
                    else accs[i] for i in range(0, len(accs), 2)]
        bv, bc, b0, b1, b2 = accs[0]

        m = jnp.max(bv, axis=1, keepdims=True)
        eq = bv == m
        n_cand = bc * _LN + lane
        base = n_cand << 16

        def keymin(payload):
            k = jnp.where(eq, base | payload, jnp.int32(0x7FFFFFFF))
            return jnp.min(k, axis=1, keepdims=True)

        def halves(v):
            vb = lax.bitcast_convert_type(v, jnp.int32)
            return (vb >> 16) & jnp.int32(0xFFFF), vb & jnp.int32(0xFFFF)

        h0, l0 = halves(b0)
        h1, l1 = halves(b1)
        h2, l2 = halves(b2)
        k0h, k0l = keymin(h0), keymin(l0)
        k1h, k1l = keymin(h1), keymin(l1)
        k2h, k2l = keymin(h2), keymin(l2)

        def coord(kh, kl):
            vb = ((kh & jnp.int32(0xFFFF)) << 16) | (kl & jnp.int32(0xFFFF))
            return lax.bitcast_convert_type(vb, jnp.float32)

        p0 = jnp.broadcast_to(coord(k0h, k0l), (_B, _LN))
        p1 = jnp.broadcast_to(coord(k1h, k1l), (_B, _LN))
        p2 = jnp.broadcast_to(coord(k2h, k2l), (_B, _LN))
        nxt = k0h >> 16
        idx_ref[pl.ds(j, 1), :] = nxt.reshape(1, _B)
        return (p0, p1, p2)

    lax.fori_loop(1, _K, body, (p0, p1, p2))


def _fps_indices(xt, interpret=False):
    return pl.pallas_call(
        _fps_body,
        out_shape=jax.ShapeDtypeStruct((_K, _B), jnp.int32),
        scratch_shapes=[pltpu.VMEM((_B, _N), jnp.float32)],
        interpret=interpret,
    )(xt)


_HALF = _K // 2


def _sc_gather_body(x_hbm, idx_hbm, out_hbm, xv, iv, ov):
    wid = lax.axis_index("s") * 2 + lax.axis_index("c")
    b = wid // 2
    h = wid % 2
    pltpu.sync_copy(x_hbm.at[b], xv)
    pltpu.sync_copy(idx_hbm.at[b, pl.ds(h * _HALF, _HALF)], iv)
    for t in range(_HALF // 16):
        base = iv[pl.ds(t * 16, 16)] * 3
        for c in range(_C):
            vals = plsc.load_gather(xv, [base + c])
            ov[c, pl.ds(t * 16, 16)] = vals
    pltpu.sync_copy(ov, out_hbm.at[b, h])


@jax.jit
def _sc_gather(xflat, idx):
    mesh = plsc.VectorSubcoreMesh(core_axis_name="c", subcore_axis_name="s")
    run = functools.partial(
        pl.kernel,
        mesh=mesh,
        out_type=jax.ShapeDtypeStruct((_B, 2, _C, _HALF), jnp.float32),
        scratch_types=[
            pltpu.VMEM((_N * _C,), jnp.float32),
            pltpu.VMEM((_HALF,), jnp.int32),
            pltpu.VMEM((_C, _HALF), jnp.float32),
        ],
        compiler_params=pltpu.CompilerParams(needs_layout_passes=False),
    )(_sc_gather_body)
    out4 = run(xflat, idx)
    return out4.transpose(0, 1, 3, 2).reshape(_B, _K, _C)


@jax.jit
def kernel(x):
    xt = jnp.transpose(x, (2, 0, 1))
    idx_kb = _fps_indices(xt)
    idx = idx_kb.T
    return _sc_gather(x.reshape(_B, _N * _C), idx)


# speedup vs baseline: 42.3972x; 1.0016x over previous
import functools

import jax
import jax.numpy as jnp
from jax import lax
from jax.experimental import pallas as pl
from jax.experimental.pallas import tpu as pltpu
from jax.experimental.pallas import tpu_sc as plsc

_B, _N, _C, _K = 16, 8192, 3, 512


_LN = 128
_CH = _N // _LN
_NS = 2
_CPS = _CH // _NS


def _fps_body(xt_ref, idx_ref, dist_ref):
    idx_ref[0:1, :] = jnp.zeros((1, _B), jnp.int32)
    dist_ref[...] = jnp.full((_B, _N), jnp.inf, jnp.float32)
    p0 = jnp.broadcast_to(xt_ref[0, :, 0:1], (_B, _LN))
    p1 = jnp.broadcast_to(xt_ref[1, :, 0:1], (_B, _LN))
    p2 = jnp.broadcast_to(xt_ref[2, :, 0:1], (_B, _LN))
    lane = lax.broadcasted_iota(jnp.int32, (_B, _LN), 1)

    def body(j, carry):
        p0, p1, p2 = carry

        def chunk_vals(c):
            sl = pl.ds(c * _LN, _LN)
            x0 = xt_ref[0, :, sl]
            x1 = xt_ref[1, :, sl]
            x2 = xt_ref[2, :, sl]
            t0 = x0 - p0
            t1 = x1 - p1
            t2 = x2 - p2
            d = t0 * t0 + t1 * t1 + t2 * t2
            nd = jnp.minimum(dist_ref[:, sl], d)
            dist_ref[:, sl] = nd
            return nd, x0, x1, x2

        accs = []
        for s in range(_NS):
            c0 = s * _CPS
            nd, x0, x1, x2 = chunk_vals(c0)
            bv, bc = nd, jnp.full((_B, _LN), c0, jnp.int32)
            b0, b1, b2 = x0, x1, x2
            for t in range(1, _CPS):
                c = c0 + t
                nd, x0, x1, x2 = chunk_vals(c)
                upd = nd > bv
                bv = jnp.where(upd, nd, bv)
                bc = jnp.where(upd, c, bc)
                b0 = jnp.where(upd, x0, b0)
                b1 = jnp.where(upd, x1, b1)
                b2 = jnp.where(upd, x2, b2)
            accs.append((bv, bc, b0, b1, b2))

        def merge(a, b):
            upd = b[0] > a[0]
            return tuple(jnp.where(upd, yb, ya) for ya, yb in zip(a, b))

        while len(accs) > 1:
            accs = [merge(accs[i], accs[i + 1]) if i + 1 < len(accs)
                    else accs[i] for i in range(0, len(accs), 2)]
        bv, bc, b0, b1, b2 = accs[0]

        m = jnp.max(bv, axis=1, keepdims=True)
        eq = bv == m
        n_cand = bc * _LN + lane
        nf = n_cand.astype(jnp.float32) * 2048.0

        def keymin(payload):
            k = jnp.where(eq, nf + payload.astype(jnp.float32), 16777216.0)
            return jnp.min(k, axis=1, keepdims=True).astype(jnp.int32)

        def chunks(v):
            vb = lax.bitcast_convert_type(v, jnp.int32)
            return ((vb >> 21) & jnp.int32(0x7FF),
                    (vb >> 10) & jnp.int32(0x7FF),
                    vb & jnp.int32(0x3FF))

        def coord(ka, kb, kc):
            vb = (((ka & jnp.int32(0x7FF)) << 21)
                  | ((kb & jnp.int32(0x7FF)) << 10)
                  | (kc & jnp.int32(0x3FF)))
            return lax.bitcast_convert_type(vb, jnp.float32)

        a0, b0c, c0 = chunks(b0)
        a1, b1c, c1 = chunks(b1)
        a2, b2c, c2 = chunks(b2)
        k0a, k0b, k0c = keymin(a0), keymin(b0c), keymin(c0)
        k1a, k1b, k1c = keymin(a1), keymin(b1c), keymin(c1)
        k2a, k2b, k2c = keymin(a2), keymin(b2c), keymin(c2)

        p0 = jnp.broadcast_to(coord(k0a, k0b, k0c), (_B, _LN))
        p1 = jnp.broadcast_to(coord(k1a, k1b, k1c), (_B, _LN))
        p2 = jnp.broadcast_to(coord(k2a, k2b, k2c), (_B, _LN))
        nxt = k0a >> 11
        idx_ref[pl.ds(j, 1), :] = nxt.reshape(1, _B)
        return (p0, p1, p2)

    lax.fori_loop(1, _K, body, (p0, p1, p2))


def _fps_indices(xt, interpret=False):
    return pl.pallas_call(
        _fps_body,
        out_shape=jax.ShapeDtypeStruct((_K, _B), jnp.int32),
        scratch_shapes=[pltpu.VMEM((_B, _N), jnp.float32)],
        interpret=interpret,
    )(xt)


_HALF = _K // 2


def _sc_gather_body(x_hbm, idx_hbm, out_hbm, xv, iv, ov):
    wid = lax.axis_index("s") * 2 + lax.axis_index("c")
    b = wid // 2
    h = wid % 2
    pltpu.sync_copy(x_hbm.at[b], xv)
    pltpu.sync_copy(idx_hbm.at[b, pl.ds(h * _HALF, _HALF)], iv)
    for t in range(_HALF // 16):
        base = iv[pl.ds(t * 16, 16)] * 3
        for c in range(_C):
            vals = plsc.load_gather(xv, [base + c])
            ov[c, pl.ds(t * 16, 16)] = vals
    pltpu.sync_copy(ov, out_hbm.at[b, h])


@jax.jit
def _sc_gather(xflat, idx):
    mesh = plsc.VectorSubcoreMesh(core_axis_name="c", subcore_axis_name="s")
    run = functools.partial(
        pl.kernel,
        mesh=mesh,
        out_type=jax.ShapeDtypeStruct((_B, 2, _C, _HALF), jnp.float32),
        scratch_types=[
            pltpu.VMEM((_N * _C,), jnp.float32),
            pltpu.VMEM((_HALF,), jnp.int32),
            pltpu.VMEM((_C, _HALF), jnp.float32),
        ],
        compiler_params=pltpu.CompilerParams(needs_layout_passes=False),
    )(_sc_gather_body)
    out4 = run(xflat, idx)
    return out4.transpose(0, 1, 3, 2).reshape(_B, _K, _C)


@jax.jit
def kernel(x):
    xt = jnp.transpose(x, (2, 0, 1))
    idx_kb = _fps_indices(xt)
    idx = idx_kb.T
    return _sc_gather(x.reshape(_B, _N * _C), idx)
